# baseline (speedup 1.0000x reference)
import jax
import jax.numpy as jnp
from jax import lax
from jax.experimental import pallas as pl
from jax.experimental.pallas import tpu as pltpu

_F_IN, _HID, _OUT = 3, 16, 7
_LB = 128
_W1B = 0
_W2B = 128
_W3S = 256
_BCOL = 384
_B3R = 512
_REP = 520
_PROWS = 648

_IN_FP, _IN_HP = 8, 128
_IN_W1, _IN_W2, _IN_W3 = 0, _IN_FP, _IN_FP + _IN_HP
_IN_B1 = _IN_FP + 2 * _IN_HP
_IN_B2 = _IN_B1 + 8
_IN_B3 = _IN_B2 + 8


def _gcn_kernel(xs_ref, r_ref, q8_ref, a8_ref, w8_ref, b_ref, p_ref, out_ref):
    e = r_ref.shape[2]
    f32, bf16 = jnp.float32, jnp.bfloat16

    xs = xs_ref[0]
    r = r_ref[0]
    q8 = q8_ref[0]
    a8 = a8_ref[0]
    w8 = w8_ref[0]
    bcol = b_ref[0]

    w1b = p_ref[_W1B:_W1B + 128, :64]
    w2b = p_ref[_W2B:_W2B + 128, :]
    w3s = p_ref[_W3S:_W3S + 128, :]
    b1s = p_ref[_BCOL:_BCOL + 128, 0:1]
    b2s = p_ref[_BCOL:_BCOL + 128, 1:2]
    b3r = p_ref[_B3R:_B3R + 1, :]
    rep = p_ref[_REP:_REP + 128, :8]

    lane_iota = lax.broadcasted_iota(jnp.int32, (e, _LB), 1)
    ohb = (lane_iota == bcol).astype(bf16)
    row_iota = lax.broadcasted_iota(jnp.int32, (_LB, e), 0)
    ohr = (row_iota == r).astype(bf16)

    qb = [(q8 & (1 << i)) != 0 for i in range(3)]
    iota8 = lax.broadcasted_iota(jnp.int32, (8, e), 0)
    am = [a8 == k for k in range(8)]

    qd = jnp.where(a8 == iota8, w8, 0.0).astype(bf16)
    deg = jnp.dot(qd, ohb, preferred_element_type=f32) + 1.0
    dinv = lax.rsqrt(deg)
    dinv_s = jnp.dot(rep, dinv, preferred_element_type=f32)
    dinv2_s = dinv_s * dinv_s

    def tree_sel(p):
        s01 = jnp.where(qb[0], p[1], p[0])
        s23 = jnp.where(qb[0], p[3], p[2])
        s45 = jnp.where(qb[0], p[5], p[4])
        s67 = jnp.where(qb[0], p[7], p[6])
        lo = jnp.where(qb[1], s23, s01)
        hi = jnp.where(qb[1], s67, s45)
        return jnp.where(qb[2], hi, lo)

    def a_hat(vt):
        vs = (vt * dinv_s).astype(bf16)
        p_all = jnp.dot(vs, ohr, preferred_element_type=f32)
        top = tree_sel([p_all[16 * k:16 * k + 8, :] for k in range(8)])
        bot = tree_sel([p_all[16 * k + 8:16 * k + 16, :] for k in range(8)])
        top = top * w8
        bot = bot * w8
        qmat = jnp.concatenate(
            [jnp.where(am[k], half, 0.0)
             for k in range(8) for half in (top, bot)],
            axis=0).astype(bf16)
        out_all = jnp.dot(qmat, ohb, preferred_element_type=f32)
        return out_all * dinv_s + vt * dinv2_s

    vt1 = jnp.dot(w1b, xs, preferred_element_type=f32)
    h1 = jnp.maximum(a_hat(vt1) + b1s, 0.0)
    vt2 = jnp.dot(w2b, h1, preferred_element_type=f32)
    h2 = jnp.maximum(a_hat(vt2) + b2s, 0.0)

    pooled = jnp.sum(h2, axis=1, keepdims=True)
    out_ref[0] = jnp.sum(pooled * w3s, axis=0, keepdims=True) + b3r


@jax.jit
def _forward(x, edge_index, edge_weight, packed_params):
    B, N, _ = x.shape
    E = edge_index.shape[2]
    nb = N // _LB

    src = edge_index[:, 0, :]
    tgt = edge_index[:, 1, :]
    r_row = (src & (_LB - 1))[:, None, :]
    q8 = jnp.broadcast_to((src >> 7)[:, None, :], (B, 8, E))
    a8 = jnp.broadcast_to((tgt >> 7)[:, None, :], (B, 8, E))
    w8 = jnp.broadcast_to(edge_weight[:, None, :], (B, 8, E))
    b_col = (tgt & (_LB - 1))[:, :, None]

    xt = jnp.zeros((B, 8, N), jnp.float32).at[:, :_F_IN, :].set(
        jnp.swapaxes(x, 1, 2))
    xs = jnp.swapaxes(xt.reshape(B, 8, nb, _LB), 1, 2).reshape(B, 64, _LB)

    pp = packed_params
    w1t = jnp.zeros((16, 8), jnp.float32).at[:, :_F_IN].set(
        jnp.swapaxes(pp[_IN_W1:_IN_W1 + _F_IN, :16], 0, 1))
    w2t = jnp.swapaxes(pp[_IN_W2:_IN_W2 + 16, :16], 0, 1)
    eye8 = jnp.eye(8, dtype=jnp.float32)
    pbuf = jnp.zeros((_PROWS, 128), jnp.float32)
    pbuf = pbuf.at[_W1B:_W1B + 128, :64].set(jnp.kron(eye8, w1t))
    pbuf = pbuf.at[_W2B:_W2B + 128, :].set(jnp.kron(eye8, w2t))
    pbuf = pbuf.at[_W3S:_W3S + 128, :].set(
        jnp.tile(pp[_IN_W3:_IN_W3 + 16, :], (8, 1)))
    pbuf = pbuf.at[_BCOL:_BCOL + 128, 0].set(jnp.tile(pp[_IN_B1, :16], 8))
    pbuf = pbuf.at[_BCOL:_BCOL + 128, 1].set(jnp.tile(pp[_IN_B2, :16], 8))
    pbuf = pbuf.at[_B3R, :].set(pp[_IN_B3, :])
    pbuf = pbuf.at[_REP:_REP + 128, :8].set(jnp.kron(eye8, jnp.ones((16, 1))))

    out = pl.pallas_call(
        _gcn_kernel,
        out_shape=jax.ShapeDtypeStruct((B, 1, 128), jnp.float32),
        grid=(B,),
        in_specs=[
            pl.BlockSpec((1, 64, _LB), lambda g: (g, 0, 0)),
            pl.BlockSpec((1, 1, E), lambda g: (g, 0, 0)),
            pl.BlockSpec((1, 8, E), lambda g: (g, 0, 0)),
            pl.BlockSpec((1, 8, E), lambda g: (g, 0, 0)),
            pl.BlockSpec((1, 8, E), lambda g: (g, 0, 0)),
            pl.BlockSpec((1, E, 1), lambda g: (g, 0, 0)),
            pl.BlockSpec((_PROWS, 128), lambda g: (0, 0)),
        ],
        out_specs=pl.BlockSpec((1, 1, 128), lambda g: (g, 0, 0)),
        compiler_params=pltpu.CompilerParams(
            dimension_semantics=("parallel",)),
    )(xs, r_row, q8, a8, w8, b_col, pbuf)

    return out[:, 0, :_OUT]


def kernel(x, edge_index, edge_weight, packed_params):
    return _forward(x, edge_index, edge_weight, packed_params)


# speedup vs baseline: 1.0085x; 1.0085x over previous
import jax
import jax.numpy as jnp
from jax import lax
from jax.experimental import pallas as pl
from jax.experimental.pallas import tpu as pltpu

_F_IN, _HID, _OUT = 3, 16, 7
_LB = 128
_W1B = 0
_W2B = 128
_W3S = 256
_BCOL = 384
_B3R = 512
_REP = 520
_PROWS = 648

_IN_FP, _IN_HP = 8, 128
_IN_W1, _IN_W2, _IN_W3 = 0, _IN_FP, _IN_FP + _IN_HP
_IN_B1 = _IN_FP + 2 * _IN_HP
_IN_B2 = _IN_B1 + 8
_IN_B3 = _IN_B2 + 8


def _gcn_kernel(xs_ref, r_ref, q8_ref, a8_ref, w8_ref, b_ref, p_ref, out_ref):
    e = r_ref.shape[2]
    f32 = jnp.float32

    xs = xs_ref[0]
    r = r_ref[0]
    q8 = q8_ref[0]
    a8 = a8_ref[0]
    w8 = w8_ref[0]
    bcol = b_ref[0]

    w1b = p_ref[_W1B:_W1B + 128, :64]
    w2b = p_ref[_W2B:_W2B + 128, :]
    w3s = p_ref[_W3S:_W3S + 128, :]
    b1s = p_ref[_BCOL:_BCOL + 128, 0:1]
    b2s = p_ref[_BCOL:_BCOL + 128, 1:2]
    b3r = p_ref[_B3R:_B3R + 1, :]
    rep = p_ref[_REP:_REP + 128, :8]

    lane_iota = lax.broadcasted_iota(jnp.int32, (e, _LB), 1)
    ohb = (lane_iota == bcol).astype(f32)
    row_iota = lax.broadcasted_iota(jnp.int32, (_LB, e), 0)
    ohrw = jnp.where(row_iota == r, w8[0:1, :], 0.0)

    qmf = [(q8 == k).astype(f32) for k in range(8)]
    amf = [(a8 == k).astype(f32) for k in range(8)]

    iota8 = lax.broadcasted_iota(jnp.int32, (8, e), 0)
    qd = jnp.where(a8 == iota8, w8, 0.0)
    deg = jnp.dot(qd, ohb, preferred_element_type=f32) + 1.0
    dinv = lax.rsqrt(deg)
    dinv_s = jnp.dot(rep, dinv, preferred_element_type=f32)
    dinv2_s = dinv_s * dinv_s

    def a_hat(vt):
        vs = vt * dinv_s
        p_all = jnp.dot(vs, ohrw, preferred_element_type=f32)
        top = p_all[0:8, :] * qmf[0]
        bot = p_all[8:16, :] * qmf[0]
        for k in range(1, 8):
            top = top + p_all[16 * k:16 * k + 8, :] * qmf[k]
            bot = bot + p_all[16 * k + 8:16 * k + 16, :] * qmf[k]
        qmat = jnp.concatenate(
            [half * amf[k] for k in range(8) for half in (top, bot)],
            axis=0)
        out_all = jnp.dot(qmat, ohb, preferred_element_type=f32)
        return out_all * dinv_s + vt * dinv2_s

    vt1 = jnp.dot(w1b, xs, preferred_element_type=f32)
    h1 = jnp.maximum(a_hat(vt1) + b1s, 0.0)
    vt2 = jnp.dot(w2b, h1, preferred_element_type=f32)
    h2 = jnp.maximum(a_hat(vt2) + b2s, 0.0)

    pooled = jnp.sum(h2, axis=1, keepdims=True)
    out_ref[0] = jnp.sum(pooled * w3s, axis=0, keepdims=True) + b3r


@jax.jit
def _forward(x, edge_index, edge_weight, packed_params):
    B, N, _ = x.shape
    E = edge_index.shape[2]
    nb = N // _LB

    src = edge_index[:, 0, :]
    tgt = edge_index[:, 1, :]
    r_row = (src & (_LB - 1))[:, None, :]
    q8 = jnp.broadcast_to((src >> 7)[:, None, :], (B, 8, E))
    a8 = jnp.broadcast_to((tgt >> 7)[:, None, :], (B, 8, E))
    w8 = jnp.broadcast_to(edge_weight[:, None, :], (B, 8, E))
    b_col = (tgt & (_LB - 1))[:, :, None]

    xt = jnp.zeros((B, 8, N), jnp.float32).at[:, :_F_IN, :].set(
        jnp.swapaxes(x, 1, 2))
    xs = jnp.swapaxes(xt.reshape(B, 8, nb, _LB), 1, 2).reshape(B, 64, _LB)

    pp = packed_params
    w1t = jnp.zeros((16, 8), jnp.float32).at[:, :_F_IN].set(
        jnp.swapaxes(pp[_IN_W1:_IN_W1 + _F_IN, :16], 0, 1))
    w2t = jnp.swapaxes(pp[_IN_W2:_IN_W2 + 16, :16], 0, 1)
    eye8 = jnp.eye(8, dtype=jnp.float32)
    pbuf = jnp.zeros((_PROWS, 128), jnp.float32)
    pbuf = pbuf.at[_W1B:_W1B + 128, :64].set(jnp.kron(eye8, w1t))
    pbuf = pbuf.at[_W2B:_W2B + 128, :].set(jnp.kron(eye8, w2t))
    pbuf = pbuf.at[_W3S:_W3S + 128, :].set(
        jnp.tile(pp[_IN_W3:_IN_W3 + 16, :], (8, 1)))
    pbuf = pbuf.at[_BCOL:_BCOL + 128, 0].set(jnp.tile(pp[_IN_B1, :16], 8))
    pbuf = pbuf.at[_BCOL:_BCOL + 128, 1].set(jnp.tile(pp[_IN_B2, :16], 8))
    pbuf = pbuf.at[_B3R, :].set(pp[_IN_B3, :])
    pbuf = pbuf.at[_REP:_REP + 128, :8].set(jnp.kron(eye8, jnp.ones((16, 1))))

    out = pl.pallas_call(
        _gcn_kernel,
        out_shape=jax.ShapeDtypeStruct((B, 1, 128), jnp.float32),
        grid=(B,),
        in_specs=[
            pl.BlockSpec((1, 64, _LB), lambda g: (g, 0, 0)),
            pl.BlockSpec((1, 1, E), lambda g: (g, 0, 0)),
            pl.BlockSpec((1, 8, E), lambda g: (g, 0, 0)),
            pl.BlockSpec((1, 8, E), lambda g: (g, 0, 0)),
            pl.BlockSpec((1, 8, E), lambda g: (g, 0, 0)),
            pl.BlockSpec((1, E, 1), lambda g: (g, 0, 0)),
            pl.BlockSpec((_PROWS, 128), lambda g: (0, 0)),
        ],
        out_specs=pl.BlockSpec((1, 1, 128), lambda g: (g, 0, 0)),
        compiler_params=pltpu.CompilerParams(
            dimension_semantics=("parallel",)),
    )(xs, r_row, q8, a8, w8, b_col, pbuf)

    return out[:, 0, :_OUT]


def kernel(x, edge_index, edge_weight, packed_params):
    return _forward(x, edge_index, edge_weight, packed_params)


# speedup vs baseline: 1.0582x; 1.0493x over previous
import jax
import jax.numpy as jnp
from jax import lax
from jax.experimental import pallas as pl
from jax.experimental.pallas import tpu as pltpu

_F_IN, _HID, _OUT = 3, 16, 7
_LB = 128
_W1B = 0
_W2B = 128
_W3S = 256
_BCOL = 384
_B3R = 512
_REP = 520
_PROWS = 648

_IN_FP, _IN_HP = 8, 128
_IN_W1, _IN_W2, _IN_W3 = 0, _IN_FP, _IN_FP + _IN_HP
_IN_B1 = _IN_FP + 2 * _IN_HP
_IN_B2 = _IN_B1 + 8
_IN_B3 = _IN_B2 + 8


def _gcn_kernel(xs_ref, r_ref, q_ref, a_ref, w_ref, b_ref, p_ref, out_ref):
    e = r_ref.shape[2]
    f32 = jnp.float32

    xs = xs_ref[0]
    r = r_ref[0]
    q = q_ref[0]
    aa = a_ref[0]
    w = w_ref[0]
    bcol = b_ref[0]

    w1b = p_ref[_W1B:_W1B + 128, :64]
    w2b = p_ref[_W2B:_W2B + 128, :]
    w3s = p_ref[_W3S:_W3S + 128, :]
    b1s = p_ref[_BCOL:_BCOL + 128, 0:1]
    b2s = p_ref[_BCOL:_BCOL + 128, 1:2]
    b3r = p_ref[_B3R:_B3R + 1, :]
    rep = p_ref[_REP:_REP + 128, :8]

    lane_iota = lax.broadcasted_iota(jnp.int32, (e, _LB), 1)
    ohb = (lane_iota == bcol).astype(f32)
    row_iota = lax.broadcasted_iota(jnp.int32, (_LB, e), 0)
    ohrw = jnp.where(row_iota == r, w, 0.0)

    qmf = [(q == k).astype(f32) for k in range(8)]
    amf = [(aa == k).astype(f32) for k in range(8)]

    iota8 = lax.broadcasted_iota(jnp.int32, (8, e), 0)
    qd = jnp.where(aa == iota8, w, 0.0)
    deg = jnp.dot(qd, ohb, preferred_element_type=f32) + 1.0
    dinv = lax.rsqrt(deg)
    dinv_s = jnp.dot(rep, dinv, preferred_element_type=f32)
    dinv2_s = dinv_s * dinv_s

    def a_hat(vt):
        vs = vt * dinv_s
        p_all = jnp.dot(vs, ohrw, preferred_element_type=f32)
        top = p_all[0:8, :] * qmf[0]
        bot = p_all[8:16, :] * qmf[0]
        for k in range(1, 8):
            top = top + p_all[16 * k:16 * k + 8, :] * qmf[k]
            bot = bot + p_all[16 * k + 8:16 * k + 16, :] * qmf[k]
        qmat = jnp.concatenate(
            [half * amf[k] for k in range(8) for half in (top, bot)],
            axis=0)
        out_all = jnp.dot(qmat, ohb, preferred_element_type=f32)
        return out_all * dinv_s + vt * dinv2_s

    vt1 = jnp.dot(w1b, xs, preferred_element_type=f32)
    h1 = jnp.maximum(a_hat(vt1) + b1s, 0.0)
    vt2 = jnp.dot(w2b, h1, preferred_element_type=f32)
    h2 = jnp.maximum(a_hat(vt2) + b2s, 0.0)

    pooled = jnp.sum(h2, axis=1, keepdims=True)
    out_ref[0] = jnp.sum(pooled * w3s, axis=0, keepdims=True) + b3r


@jax.jit
def _forward(x, edge_index, edge_weight, packed_params):
    B, N, _ = x.shape
    E = edge_index.shape[2]
    nb = N // _LB

    src = edge_index[:, 0, :]
    tgt = edge_index[:, 1, :]
    r_row = (src & (_LB - 1))[:, None, :]
    q_row = (src >> 7)[:, None, :]
    a_row = (tgt >> 7)[:, None, :]
    w_row = edge_weight[:, None, :]
    b_col = (tgt & (_LB - 1))[:, :, None]

    xt = jnp.zeros((B, 8, N), jnp.float32).at[:, :_F_IN, :].set(
        jnp.swapaxes(x, 1, 2))
    xs = jnp.swapaxes(xt.reshape(B, 8, nb, _LB), 1, 2).reshape(B, 64, _LB)

    pp = packed_params
    w1t = jnp.zeros((16, 8), jnp.float32).at[:, :_F_IN].set(
        jnp.swapaxes(pp[_IN_W1:_IN_W1 + _F_IN, :16], 0, 1))
    w2t = jnp.swapaxes(pp[_IN_W2:_IN_W2 + 16, :16], 0, 1)
    eye8 = jnp.eye(8, dtype=jnp.float32)
    pbuf = jnp.zeros((_PROWS, 128), jnp.float32)
    pbuf = pbuf.at[_W1B:_W1B + 128, :64].set(jnp.kron(eye8, w1t))
    pbuf = pbuf.at[_W2B:_W2B + 128, :].set(jnp.kron(eye8, w2t))
    pbuf = pbuf.at[_W3S:_W3S + 128, :].set(
        jnp.tile(pp[_IN_W3:_IN_W3 + 16, :], (8, 1)))
    pbuf = pbuf.at[_BCOL:_BCOL + 128, 0].set(jnp.tile(pp[_IN_B1, :16], 8))
    pbuf = pbuf.at[_BCOL:_BCOL + 128, 1].set(jnp.tile(pp[_IN_B2, :16], 8))
    pbuf = pbuf.at[_B3R, :].set(pp[_IN_B3, :])
    pbuf = pbuf.at[_REP:_REP + 128, :8].set(jnp.kron(eye8, jnp.ones((16, 1))))

    out = pl.pallas_call(
        _gcn_kernel,
        out_shape=jax.ShapeDtypeStruct((B, 1, 128), jnp.float32),
        grid=(B,),
        in_specs=[
            pl.BlockSpec((1, 64, _LB), lambda g: (g, 0, 0)),
            pl.BlockSpec((1, 1, E), lambda g: (g, 0, 0)),
            pl.BlockSpec((1, 1, E), lambda g: (g, 0, 0)),
            pl.BlockSpec((1, 1, E), lambda g: (g, 0, 0)),
            pl.BlockSpec((1, 1, E), lambda g: (g, 0, 0)),
            pl.BlockSpec((1, E, 1), lambda g: (g, 0, 0)),
            pl.BlockSpec((_PROWS, 128), lambda g: (0, 0)),
        ],
        out_specs=pl.BlockSpec((1, 1, 128), lambda g: (g, 0, 0)),
        compiler_params=pltpu.CompilerParams(
            dimension_semantics=("parallel",)),
    )(xs, r_row, q_row, a_row, w_row, b_col, pbuf)

    return out[:, 0, :_OUT]


def kernel(x, edge_index, edge_weight, packed_params):
    return _forward(x, edge_index, edge_weight, packed_params)
